# Initial kernel scaffold; baseline (speedup 1.0000x reference)
import jax
import jax.numpy as jnp
from jax.experimental import pallas as pl


def kernel(feat0, feat1, feat2, targets):
    raise NotImplementedError("write your pallas kernel here")


# speedup vs baseline: 2.7798x; 2.7798x over previous
import functools

import jax
import jax.numpy as jnp
from jax.experimental import pallas as pl

_STRIDES = (8.0, 16.0, 32.0)
_HW = ((64, 64), (32, 32), (16, 16))
_NS = tuple(h * w for h, w in _HW)
_OFFS = (0, _NS[0], _NS[0] + _NS[1])
_NTOT = sum(_NS)
_C = 85


def _decode_level(v, stride, w):
    n = v.shape[0]
    p = jax.lax.broadcasted_iota(jnp.int32, (n, 1), 0)
    gx = (p % w).astype(jnp.float32)
    gy = (p // w).astype(jnp.float32)
    c = jax.lax.broadcasted_iota(jnp.int32, (n, _C), 1)
    g = jnp.where(c == 0, gx, gy)
    xy = (v + g) * stride
    wh = jnp.exp(v) * stride
    return jnp.where(c < 2, xy, jnp.where(c < 4, wh, v))


def _decode_kernel(f0_ref, f1_ref, f2_ref, out_ref):
    for ref, stride, (h, w), off, n in zip(
        (f0_ref, f1_ref, f2_ref), _STRIDES, _HW, _OFFS, _NS
    ):
        out_ref[0, pl.ds(off, n), :] = _decode_level(ref[0], stride, w)


@jax.jit
def kernel(feat0, feat1, feat2, targets):
    b = feat0.shape[0]
    f0 = feat0.reshape(b, _NS[0], _C)
    f1 = feat1.reshape(b, _NS[1], _C)
    f2 = feat2.reshape(b, _NS[2], _C)
    return pl.pallas_call(
        _decode_kernel,
        grid=(b,),
        in_specs=[
            pl.BlockSpec((1, _NS[0], _C), lambda i: (i, 0, 0)),
            pl.BlockSpec((1, _NS[1], _C), lambda i: (i, 0, 0)),
            pl.BlockSpec((1, _NS[2], _C), lambda i: (i, 0, 0)),
        ],
        out_specs=pl.BlockSpec((1, _NTOT, _C), lambda i: (i, 0, 0)),
        out_shape=jax.ShapeDtypeStruct((b, _NTOT, _C), jnp.float32),
    )(f0, f1, f2)
